# Initial kernel scaffold; baseline (speedup 1.0000x reference)
import jax
import jax.numpy as jnp
from jax.experimental import pallas as pl


def kernel(x, table):
    raise NotImplementedError("write your pallas kernel here")


# speedup vs baseline: 1.2008x; 1.2008x over previous
import functools
import math

import jax
import jax.numpy as jnp
import numpy as np
from jax import lax
from jax.experimental import pallas as pl
from jax.experimental.pallas import tpu as pltpu
from jax.experimental.pallas import tpu_sc as plsc

_VOCAB = 1000000
_D = 32
_W = 200
_B = 4096

_NW = 32
_ROWS = _B * _W
_RPW = _ROWS // _NW
_CHUNK = 128
_NCHUNK = _RPW // _CHUNK
_NBUF = 8
_NGROUP = _NCHUNK // _NBUF

_SCALE = math.sqrt(float(_D))


def _pe_dup() -> np.ndarray:
    half = _D / 2
    positions = np.arange(_W)[:, np.newaxis]
    depths = np.arange(half)[np.newaxis, :] / half
    angle_rads = positions * (1.0 / 10000**depths)
    pe = np.concatenate([np.sin(angle_rads), np.cos(angle_rads)], axis=-1)
    pe = pe.astype(np.float32)
    return np.concatenate([pe, pe], axis=0)


_PE_DUP = _pe_dup()


def _make_sc_call():
    mesh = plsc.VectorSubcoreMesh(core_axis_name="c", subcore_axis_name="s")

    scratch = [
        pltpu.VMEM((_RPW,), jnp.int32),
        pltpu.VMEM((2 * _W, _D), jnp.float32),
    ]
    scratch += [pltpu.VMEM((_CHUNK, _D), jnp.float32) for _ in range(_NBUF)]
    scratch += [pltpu.SemaphoreType.DMA for _ in range(2 * _NBUF)]

    @functools.partial(
        pl.kernel,
        mesh=mesh,
        out_type=jax.ShapeDtypeStruct((_ROWS, _D), jnp.float32),
        scratch_types=scratch,
        compiler_params=pltpu.CompilerParams(use_tc_tiling_on_sc=False),
    )
    def k(table_hbm, x_hbm, pe_hbm, out_hbm, idx_v, pe_v, *rest):
        bufs = rest[:_NBUF]
        gsems = rest[_NBUF : 2 * _NBUF]
        osems = rest[2 * _NBUF :]

        wid = lax.axis_index("s") * 2 + lax.axis_index("c")
        base = wid * _RPW

        pltpu.sync_copy(pe_hbm, pe_v)
        pltpu.sync_copy(x_hbm.at[pl.ds(base, _RPW)], idx_v)

        def compute(buf, w0):
            def row_body(r, _):
                p = w0 + r
                a0 = buf[r, pl.ds(0, 16)]
                a1 = buf[r, pl.ds(16, 16)]
                buf[r, pl.ds(0, 16)] = a0 * _SCALE + pe_v[p, pl.ds(0, 16)]
                buf[r, pl.ds(16, 16)] = a1 * _SCALE + pe_v[p, pl.ds(16, 16)]
                return 0

            lax.fori_loop(0, _CHUNK, row_body, 0)

        def group_body(g, _):
            handles = []
            for j in range(_NBUF):
                c = g * _NBUF + j

                @pl.when(g > 0)
                def _wait_prev(j=j):
                    pltpu.make_async_copy(
                        bufs[j], out_hbm.at[pl.ds(0, _CHUNK)], osems[j]
                    ).wait()

                handles.append(
                    pltpu.async_copy(
                        table_hbm.at[idx_v.at[pl.ds(c * _CHUNK, _CHUNK)]],
                        bufs[j],
                        gsems[j],
                    )
                )

            for j in range(_NBUF):
                c = g * _NBUF + j
                handles[j].wait()
                compute(bufs[j], lax.rem(c * _CHUNK, _W))
                pltpu.async_copy(
                    bufs[j],
                    out_hbm.at[pl.ds(base + c * _CHUNK, _CHUNK)],
                    osems[j],
                )
            return 0

        lax.fori_loop(0, _NGROUP, group_body, 0)

        for j in range(_NBUF):
            pltpu.make_async_copy(
                bufs[j], out_hbm.at[pl.ds(0, _CHUNK)], osems[j]
            ).wait()

    return k


_SC_CALL = _make_sc_call()


@jax.jit
def kernel(x, table):
    x_flat = jnp.reshape(x, (-1,)).astype(jnp.int32)
    pe = jnp.asarray(_PE_DUP)
    out = _SC_CALL(table, x_flat, pe)
    return jnp.reshape(out, (_B, _W, _D))


# speedup vs baseline: 1.4526x; 1.2097x over previous
import functools
import math

import jax
import jax.numpy as jnp
import numpy as np
from jax import lax
from jax.experimental import pallas as pl
from jax.experimental.pallas import tpu as pltpu
from jax.experimental.pallas import tpu_sc as plsc

_VOCAB = 1000000
_D = 32
_W = 200
_B = 4096

_NW = 32
_ROWS = _B * _W
_RPW = _ROWS // _NW
_CHUNK = 128
_NCHUNK = _RPW // _CHUNK
_NBUF = 8
_NGROUP = _NCHUNK // _NBUF

_SCALE = math.sqrt(float(_D))


def _pe_dup() -> np.ndarray:
    half = _D / 2
    positions = np.arange(_W)[:, np.newaxis]
    depths = np.arange(half)[np.newaxis, :] / half
    angle_rads = positions * (1.0 / 10000**depths)
    pe = np.concatenate([np.sin(angle_rads), np.cos(angle_rads)], axis=-1)
    pe = pe.astype(np.float32)
    return np.concatenate([pe, pe], axis=0)


_PE_DUP = _pe_dup()


def _make_sc_call():
    mesh = plsc.VectorSubcoreMesh(core_axis_name="c", subcore_axis_name="s")

    scratch = [
        pltpu.VMEM((_RPW,), jnp.int32),
        pltpu.VMEM((2 * _W, _D), jnp.float32),
    ]
    scratch += [pltpu.VMEM((_CHUNK, _D), jnp.float32) for _ in range(_NBUF)]
    scratch += [pltpu.SemaphoreType.DMA for _ in range(2 * _NBUF)]

    @functools.partial(
        pl.kernel,
        mesh=mesh,
        out_type=jax.ShapeDtypeStruct((_ROWS, _D), jnp.float32),
        scratch_types=scratch,
        compiler_params=pltpu.CompilerParams(use_tc_tiling_on_sc=False),
    )
    def k(table_hbm, x_hbm, pe_hbm, out_hbm, idx_v, pe_v, *rest):
        bufs = rest[:_NBUF]
        gsems = rest[_NBUF : 2 * _NBUF]
        osems = rest[2 * _NBUF :]

        wid = lax.axis_index("s") * 2 + lax.axis_index("c")
        base = wid * _RPW

        pltpu.sync_copy(pe_hbm, pe_v)
        pltpu.sync_copy(x_hbm.at[pl.ds(base, _RPW)], idx_v)

        def compute(buf, w0):
            @plsc.parallel_loop(0, _CHUNK, step=1, unroll=8)
            def _row_body(r):
                p = w0 + r
                a0 = buf[r, pl.ds(0, 16)]
                a1 = buf[r, pl.ds(16, 16)]
                buf[r, pl.ds(0, 16)] = a0 * _SCALE + pe_v[p, pl.ds(0, 16)]
                buf[r, pl.ds(16, 16)] = a1 * _SCALE + pe_v[p, pl.ds(16, 16)]

        def group_body(g, _):
            handles = []
            for j in range(_NBUF):
                c = g * _NBUF + j

                @pl.when(g > 0)
                def _wait_prev(j=j):
                    pltpu.make_async_copy(
                        bufs[j], out_hbm.at[pl.ds(0, _CHUNK)], osems[j]
                    ).wait()

                handles.append(
                    pltpu.async_copy(
                        table_hbm.at[idx_v.at[pl.ds(c * _CHUNK, _CHUNK)]],
                        bufs[j],
                        gsems[j],
                    )
                )

            for j in range(_NBUF):
                c = g * _NBUF + j
                handles[j].wait()
                compute(bufs[j], lax.rem(c * _CHUNK, _W))
                pltpu.async_copy(
                    bufs[j],
                    out_hbm.at[pl.ds(base + c * _CHUNK, _CHUNK)],
                    osems[j],
                )
            return 0

        lax.fori_loop(0, _NGROUP, group_body, 0)

        for j in range(_NBUF):
            pltpu.make_async_copy(
                bufs[j], out_hbm.at[pl.ds(0, _CHUNK)], osems[j]
            ).wait()

    return k


_SC_CALL = _make_sc_call()


@jax.jit
def kernel(x, table):
    x_flat = jnp.reshape(x, (-1,)).astype(jnp.int32)
    pe = jnp.asarray(_PE_DUP)
    out = _SC_CALL(table, x_flat, pe)
    return jnp.reshape(out, (_B, _W, _D))


# speedup vs baseline: 1.5288x; 1.0525x over previous
import functools
import math

import jax
import jax.numpy as jnp
import numpy as np
from jax import lax
from jax.experimental import pallas as pl
from jax.experimental.pallas import tpu as pltpu
from jax.experimental.pallas import tpu_sc as plsc

_VOCAB = 1000000
_D = 32
_W = 200
_B = 4096

_NW = 32
_BLK = 128
_JB = _B // _BLK
_NBLK = _W * _JB
_BPW = _NBLK // _NW
_NBUF = 8
_NGROUP = _BPW // _NBUF

_SCALE = math.sqrt(float(_D))


def _pe() -> np.ndarray:
    half = _D / 2
    positions = np.arange(_W)[:, np.newaxis]
    depths = np.arange(half)[np.newaxis, :] / half
    angle_rads = positions * (1.0 / 10000**depths)
    pe = np.concatenate([np.sin(angle_rads), np.cos(angle_rads)], axis=-1)
    return pe.astype(np.float32)


_PE = _pe()


def _make_sc_call():
    mesh = plsc.VectorSubcoreMesh(core_axis_name="c", subcore_axis_name="s")

    scratch = [
        pltpu.VMEM((_BPW * _BLK,), jnp.int32),
        pltpu.VMEM((_W, _D), jnp.float32),
    ]
    scratch += [pltpu.VMEM((_BLK, _D), jnp.float32) for _ in range(_NBUF)]
    scratch += [pltpu.VMEM((_D, _BLK), jnp.float32) for _ in range(_NBUF)]
    scratch += [pltpu.SemaphoreType.DMA for _ in range(2 * _NBUF)]

    @functools.partial(
        pl.kernel,
        mesh=mesh,
        out_type=jax.ShapeDtypeStruct((_W, _D // 8, _JB, 8, _BLK), jnp.float32),
        scratch_types=scratch,
        compiler_params=pltpu.CompilerParams(
            use_tc_tiling_on_sc=False, needs_layout_passes=False
        ),
    )
    def k(table_hbm, xt_hbm, pe_hbm, out_hbm, idx_v, pe_v, *rest):
        rows_bufs = rest[:_NBUF]
        blk_bufs = rest[_NBUF : 2 * _NBUF]
        gsems = rest[2 * _NBUF : 3 * _NBUF]
        osems = rest[3 * _NBUF :]

        wid = lax.axis_index("s") * 2 + lax.axis_index("c")
        base = wid * _BPW

        pltpu.sync_copy(pe_hbm, pe_v)
        pltpu.sync_copy(xt_hbm.at[pl.ds(base * _BLK, _BPW * _BLK)], idx_v)

        lane = lax.iota(jnp.int32, 16)

        def compute(rows, blk, w):
            pe0 = pe_v[w, pl.ds(0, 16)]
            pe1 = pe_v[w, pl.ds(16, 16)]

            @plsc.parallel_loop(0, _BLK, step=1, unroll=4, carry=(pe0, pe1))
            def _col_body(c, carry):
                p0, p1 = carry
                col = jnp.broadcast_to(c, (16,)).astype(jnp.int32)
                v0 = rows[c, pl.ds(0, 16)] * _SCALE + p0
                v1 = rows[c, pl.ds(16, 16)] * _SCALE + p1
                plsc.store_scatter(blk, [lane, col], v0)
                plsc.store_scatter(blk, [lane + 16, col], v1)
                return carry

        def group_body(g, _):
            handles = []
            for u in range(_NBUF):
                l = g * _NBUF + u
                gid = base + l

                @pl.when(g > 0)
                def _wait_prev(u=u):
                    for i in range(4):
                        pltpu.make_async_copy(
                            blk_bufs[u].at[pl.ds(8 * i, 8), :],
                            out_hbm.at[0, i, 0],
                            osems[u],
                        ).wait()

                handles.append(
                    pltpu.async_copy(
                        table_hbm.at[idx_v.at[pl.ds(l * _BLK, _BLK)]],
                        rows_bufs[u],
                        gsems[u],
                    )
                )

            for u in range(_NBUF):
                l = g * _NBUF + u
                gid = base + l
                w = gid // _JB
                j = lax.rem(gid, _JB)
                handles[u].wait()
                compute(rows_bufs[u], blk_bufs[u], w)
                for i in range(4):
                    pltpu.async_copy(
                        blk_bufs[u].at[pl.ds(8 * i, 8), :],
                        out_hbm.at[w, i, j],
                        osems[u],
                    )
            return 0

        lax.fori_loop(0, _NGROUP, group_body, 0)

        for u in range(_NBUF):
            for i in range(4):
                pltpu.make_async_copy(
                    blk_bufs[u].at[pl.ds(8 * i, 8), :],
                    out_hbm.at[0, i, 0],
                    osems[u],
                ).wait()

    return k


_SC_CALL = _make_sc_call()


@jax.jit
def kernel(x, table):
    xt_flat = jnp.reshape(jnp.transpose(x), (-1,)).astype(jnp.int32)
    pe = jnp.asarray(_PE)
    out5 = _SC_CALL(table, xt_flat, pe)
    return jnp.reshape(jnp.transpose(out5, (2, 4, 0, 1, 3)), (_B, _W, _D))


# speedup vs baseline: 2.4648x; 1.6122x over previous
import functools
import math

import jax
import jax.numpy as jnp
import numpy as np
from jax import lax
from jax.experimental import pallas as pl
from jax.experimental.pallas import tpu as pltpu
from jax.experimental.pallas import tpu_sc as plsc

_VOCAB = 1000000
_D = 32
_W = 200
_B = 4096

_NW = 32
_BLK = 128
_JB = _B // _BLK
_NBLK = _W * _JB
_BPW = _NBLK // _NW
_NBUF = 8
_NGROUP = _BPW // _NBUF
_BLKP = _BLK + 1

_SCALE = math.sqrt(float(_D))


def _pe() -> np.ndarray:
    half = _D / 2
    positions = np.arange(_W)[:, np.newaxis]
    depths = np.arange(half)[np.newaxis, :] / half
    angle_rads = positions * (1.0 / 10000**depths)
    pe = np.concatenate([np.sin(angle_rads), np.cos(angle_rads)], axis=-1)
    return pe.astype(np.float32)


_PE = _pe()


def _make_sc_call():
    mesh = plsc.VectorSubcoreMesh(core_axis_name="c", subcore_axis_name="s")

    scratch = [
        pltpu.VMEM((_BPW * _BLK,), jnp.int32),
        pltpu.VMEM((_W, _D), jnp.float32),
    ]
    scratch += [pltpu.VMEM((_BLK, _D), jnp.float32) for _ in range(_NBUF)]
    scratch += [pltpu.VMEM((_D, _BLKP), jnp.float32) for _ in range(_NBUF)]
    scratch += [pltpu.SemaphoreType.DMA for _ in range(2 * _NBUF)]

    @functools.partial(
        pl.kernel,
        mesh=mesh,
        out_type=jax.ShapeDtypeStruct((_W, _D // 8, _JB, 8, _BLK), jnp.float32),
        scratch_types=scratch,
        compiler_params=pltpu.CompilerParams(
            use_tc_tiling_on_sc=False, needs_layout_passes=False
        ),
    )
    def k(table_hbm, xt_hbm, pe_hbm, out_hbm, idx_v, pe_v, *rest):
        rows_bufs = rest[:_NBUF]
        blk_bufs = rest[_NBUF : 2 * _NBUF]
        gsems = rest[2 * _NBUF : 3 * _NBUF]
        osems = rest[3 * _NBUF :]

        wid = lax.axis_index("s") * 2 + lax.axis_index("c")
        base = wid * _BPW

        pltpu.sync_copy(pe_hbm, pe_v)
        pltpu.sync_copy(xt_hbm.at[pl.ds(base * _BLK, _BPW * _BLK)], idx_v)

        lane = lax.iota(jnp.int32, 16)

        def compute(rows, blk, w):
            pe0 = pe_v[w, pl.ds(0, 16)]
            pe1 = pe_v[w, pl.ds(16, 16)]

            @plsc.parallel_loop(0, _BLK, step=1, unroll=4, carry=(pe0, pe1))
            def _col_body(c, carry):
                p0, p1 = carry
                col = jnp.broadcast_to(c, (16,)).astype(jnp.int32)
                v0 = rows[c, pl.ds(0, 16)] * _SCALE + p0
                v1 = rows[c, pl.ds(16, 16)] * _SCALE + p1
                plsc.store_scatter(blk, [lane, col], v0)
                plsc.store_scatter(blk, [lane + 16, col], v1)
                return carry

        def group_body(g, _):
            handles = []
            for u in range(_NBUF):
                l = g * _NBUF + u
                gid = base + l

                @pl.when(g > 0)
                def _wait_prev(u=u):
                    for i in range(4):
                        pltpu.make_async_copy(
                            blk_bufs[u].at[pl.ds(8 * i, 8), pl.ds(0, _BLK)],
                            out_hbm.at[0, i, 0],
                            osems[u],
                        ).wait()

                handles.append(
                    pltpu.async_copy(
                        table_hbm.at[idx_v.at[pl.ds(l * _BLK, _BLK)]],
                        rows_bufs[u],
                        gsems[u],
                    )
                )

            for u in range(_NBUF):
                l = g * _NBUF + u
                gid = base + l
                w = gid // _JB
                j = lax.rem(gid, _JB)
                handles[u].wait()
                compute(rows_bufs[u], blk_bufs[u], w)
                for i in range(4):
                    pltpu.async_copy(
                        blk_bufs[u].at[pl.ds(8 * i, 8), pl.ds(0, _BLK)],
                        out_hbm.at[w, i, j],
                        osems[u],
                    )
            return 0

        lax.fori_loop(0, _NGROUP, group_body, 0)

        for u in range(_NBUF):
            for i in range(4):
                pltpu.make_async_copy(
                    blk_bufs[u].at[pl.ds(8 * i, 8), pl.ds(0, _BLK)],
                    out_hbm.at[0, i, 0],
                    osems[u],
                ).wait()

    return k


_SC_CALL = _make_sc_call()


@jax.jit
def kernel(x, table):
    xt_flat = jnp.reshape(jnp.transpose(x), (-1,)).astype(jnp.int32)
    pe = jnp.asarray(_PE)
    out5 = _SC_CALL(table, xt_flat, pe)
    return jnp.reshape(jnp.transpose(out5, (2, 4, 0, 1, 3)), (_B, _W, _D))
